# Initial kernel scaffold; baseline (speedup 1.0000x reference)
import jax
import jax.numpy as jnp
from jax.experimental import pallas as pl


def kernel(x, edge_index, W1, b1, W_ih1, b_ih1, W_hh1, b_hh1, W2, b2, W_ih2, b_ih2, W_hh2, b_hh2, Wfc, bfc):
    raise NotImplementedError("write your pallas kernel here")


# speedup vs baseline: 4.1560x; 4.1560x over previous
import functools

import jax
import jax.numpy as jnp
from jax import lax
from jax.experimental import pallas as pl
from jax.experimental.pallas import tpu as pltpu
from jax.experimental.pallas import tpu_sc as plsc

N = 10000
E = 320000
D = 128
NSTEPS = 5

NC = 2
NS = 16
EPT = E // (NC * NS)
CHUNK = 80
NCHUNK = EPT // CHUNK
ROWS_PT = 624
ZR = 208
NZ = ROWS_PT // ZR
TAIL = N - NS * ROWS_PT


_sc_mesh = plsc.VectorSubcoreMesh(core_axis_name="c", subcore_axis_name="s")


@functools.partial(
    pl.kernel,
    out_type=jax.ShapeDtypeStruct((NC, N, D), jnp.float32),
    mesh=_sc_mesh,
    scratch_types=[
        pltpu.VMEM_SHARED((N, D), jnp.float32),
        pltpu.VMEM((CHUNK,), jnp.int32),
        pltpu.VMEM((CHUNK,), jnp.int32),
        pltpu.VMEM((CHUNK, D), jnp.float32),
        pltpu.VMEM((ZR, D), jnp.float32),
        pltpu.SemaphoreType.DMA,
    ],
)
def _segsum(m_hbm, src_hbm, dst_hbm, out_hbm, a_sh, src_v, dst_v, rows_v, zbuf_v, sem):
    cid = lax.axis_index("c")
    sid = lax.axis_index("s")

    zero16 = jnp.zeros((16,), jnp.float32)

    def zstore(i, _):
        zbuf_v[i // 8, pl.ds((i % 8) * 16, 16)] = zero16
        return 0

    lax.fori_loop(0, ZR * 8, zstore, 0)

    r0 = sid * ROWS_PT

    def zcopy(j, _):
        pltpu.sync_copy(zbuf_v, a_sh.at[pl.ds(r0 + j * ZR, ZR)])
        return 0

    lax.fori_loop(0, NZ, zcopy, 0)

    @pl.when(sid == 0)
    def _():
        pltpu.sync_copy(zbuf_v.at[pl.ds(0, TAIL)], a_sh.at[pl.ds(NS * ROWS_PT, TAIL)])

    plsc.subcore_barrier()

    base = (cid * NS + sid) * EPT

    def body(i, _):
        off = base + i * CHUNK
        pltpu.sync_copy(src_hbm.at[pl.ds(off, CHUNK)], src_v)
        pltpu.sync_copy(dst_hbm.at[pl.ds(off, CHUNK)], dst_v)
        pltpu.async_copy(m_hbm.at[src_v], rows_v, sem).wait()
        pltpu.sync_copy(rows_v, a_sh.at[dst_v], add=True)
        return 0

    lax.fori_loop(0, NCHUNK, body, 0)
    plsc.subcore_barrier()

    def wcopy(j, _):
        pltpu.sync_copy(a_sh.at[pl.ds(r0 + j * ZR, ZR)], zbuf_v)
        pltpu.sync_copy(zbuf_v, out_hbm.at[cid, pl.ds(r0 + j * ZR, ZR)])
        return 0

    lax.fori_loop(0, NZ, wcopy, 0)

    @pl.when(sid == 0)
    def _():
        pltpu.sync_copy(a_sh.at[pl.ds(NS * ROWS_PT, TAIL)], zbuf_v.at[pl.ds(0, TAIL)])
        pltpu.sync_copy(zbuf_v.at[pl.ds(0, TAIL)], out_hbm.at[cid, pl.ds(NS * ROWS_PT, TAIL)])


_BLK = 1000
_GRID = N // _BLK


def _mm_body(h_ref, w_ref, b_ref, out_ref):
    out_ref[...] = (
        jnp.dot(h_ref[...], w_ref[...], preferred_element_type=jnp.float32)
        + b_ref[...]
    )


def _mm(h, wT, b2d):
    dout = wT.shape[1]
    return pl.pallas_call(
        _mm_body,
        grid=(_GRID,),
        in_specs=[
            pl.BlockSpec((_BLK, D), lambda i: (i, 0)),
            pl.BlockSpec((D, dout), lambda i: (0, 0)),
            pl.BlockSpec((1, dout), lambda i: (0, 0)),
        ],
        out_specs=pl.BlockSpec((_BLK, dout), lambda i: (i, 0)),
        out_shape=jax.ShapeDtypeStruct((N, dout), jnp.float32),
    )(h, wT, b2d)


def _gru_body(a0_ref, a1_ref, h_ref, wih_ref, bih_ref, whh_ref, bhh_ref, out_ref):
    a = a0_ref[...] + a1_ref[...]
    h = h_ref[...]
    gi = jnp.dot(a, wih_ref[...], preferred_element_type=jnp.float32) + bih_ref[...]
    gh = jnp.dot(h, whh_ref[...], preferred_element_type=jnp.float32) + bhh_ref[...]
    r = jax.nn.sigmoid(gi[:, :D] + gh[:, :D])
    z = jax.nn.sigmoid(gi[:, D:2 * D] + gh[:, D:2 * D])
    n = jnp.tanh(gi[:, 2 * D:] + r * gh[:, 2 * D:])
    out_ref[...] = (1.0 - z) * n + z * h


def _gru(parts, h, wihT, bih2, whhT, bhh2):
    return pl.pallas_call(
        _gru_body,
        grid=(_GRID,),
        in_specs=[
            pl.BlockSpec((_BLK, D), lambda i: (i, 0)),
            pl.BlockSpec((_BLK, D), lambda i: (i, 0)),
            pl.BlockSpec((_BLK, D), lambda i: (i, 0)),
            pl.BlockSpec((D, 3 * D), lambda i: (0, 0)),
            pl.BlockSpec((1, 3 * D), lambda i: (0, 0)),
            pl.BlockSpec((D, 3 * D), lambda i: (0, 0)),
            pl.BlockSpec((1, 3 * D), lambda i: (0, 0)),
        ],
        out_specs=pl.BlockSpec((_BLK, D), lambda i: (i, 0)),
        out_shape=jax.ShapeDtypeStruct((N, D), jnp.float32),
    )(parts[0], parts[1], h, wihT, bih2, whhT, bhh2)


def kernel(x, edge_index, W1, b1, W_ih1, b_ih1, W_hh1, b_hh1,
           W2, b2, W_ih2, b_ih2, W_hh2, b_hh2, Wfc, bfc):
    src = edge_index[0]
    dst = edge_index[1]
    h = x
    for (W, b, W_ih, b_ih, W_hh, b_hh) in (
        (W1, b1, W_ih1, b_ih1, W_hh1, b_hh1),
        (W2, b2, W_ih2, b_ih2, W_hh2, b_hh2),
    ):
        wT = W.T
        b2d = b[None, :]
        wihT = W_ih.T
        bih2 = b_ih[None, :]
        whhT = W_hh.T
        bhh2 = b_hh[None, :]
        for _ in range(NSTEPS):
            m = _mm(h, wT, b2d)
            parts = _segsum(m, src, dst)
            h = _gru(parts, h, wihT, bih2, whhT, bhh2)

    wfcT = jnp.zeros((D, D), jnp.float32).at[:, :2].set(Wfc.T)
    bfc2 = jnp.zeros((1, D), jnp.float32).at[0, :2].set(bfc)
    out = _mm(h, wfcT, bfc2)
    return out[:, :2]
